# baseline (speedup 1.0000x reference)
import functools

import jax
import jax.numpy as jnp
from jax import lax
from jax.experimental import pallas as pl
from jax.experimental.pallas import tpu as pltpu
from jax.experimental.pallas import tpu_sc as plsc

N = 10000
E = 320000
IN = 128
H = 8
D = 64
HID = 128
P = 2

NT = 16
ER_ROWS = E // 128
NSA = 624


def _ka_body(h_ref, w0_ref, w1_ref, alm0_ref, arm0_ref, alm1_ref, arm1_ref,
             f0_ref, f1_ref, el0_ref, er0_ref, el1_ref, er1_ref,
             mel_ref, mer_ref):
    i = pl.program_id(0)
    hb = h_ref[...]
    f0 = jnp.dot(hb, w0_ref[...], preferred_element_type=jnp.float32)
    f1 = jnp.dot(hb, w1_ref[...], preferred_element_type=jnp.float32)
    f0_ref[...] = f0
    f1_ref[...] = f1
    el0 = jnp.dot(f0, alm0_ref[...], preferred_element_type=jnp.float32)
    er0 = jnp.dot(f0, arm0_ref[...], preferred_element_type=jnp.float32)
    el1 = jnp.dot(f1, alm1_ref[...], preferred_element_type=jnp.float32)
    er1 = jnp.dot(f1, arm1_ref[...], preferred_element_type=jnp.float32)
    el0_ref[...] = el0
    er0_ref[...] = er0
    el1_ref[...] = el1
    er1_ref[...] = er1

    @pl.when(i == 0)
    def _():
        mel_ref[...] = jnp.full((P, 8), -1e30, jnp.float32)
        mer_ref[...] = jnp.full((P, 8), -1e30, jnp.float32)

    mel_new = jnp.stack([jnp.full((8,), jnp.max(el0), jnp.float32),
                         jnp.full((8,), jnp.max(el1), jnp.float32)])
    mer_new = jnp.stack([jnp.full((8,), jnp.max(er0), jnp.float32),
                         jnp.full((8,), jnp.max(er1), jnp.float32)])
    mel_ref[...] = jnp.maximum(mel_ref[...], mel_new)
    mer_ref[...] = jnp.maximum(mer_ref[...], mer_new)


def _kernel_a(h, W0, W1, alm0, arm0, alm1, arm1):
    BN = 2000
    grid = (N // BN,)
    full = lambda i: (0, 0)
    return pl.pallas_call(
        _ka_body,
        grid=grid,
        in_specs=[
            pl.BlockSpec((BN, IN), lambda i: (i, 0)),
            pl.BlockSpec((IN, H * D), full),
            pl.BlockSpec((IN, H * D), full),
            pl.BlockSpec((H * D, 8), full),
            pl.BlockSpec((H * D, 8), full),
            pl.BlockSpec((H * D, 8), full),
            pl.BlockSpec((H * D, 8), full),
        ],
        out_specs=[
            pl.BlockSpec((BN, H * D), lambda i: (i, 0)),
            pl.BlockSpec((BN, H * D), lambda i: (i, 0)),
            pl.BlockSpec((BN, 8), lambda i: (i, 0)),
            pl.BlockSpec((BN, 8), lambda i: (i, 0)),
            pl.BlockSpec((BN, 8), lambda i: (i, 0)),
            pl.BlockSpec((BN, 8), lambda i: (i, 0)),
            pl.BlockSpec((P, 8), full),
            pl.BlockSpec((P, 8), full),
        ],
        out_shape=[
            jax.ShapeDtypeStruct((N, H * D), jnp.float32),
            jax.ShapeDtypeStruct((N, H * D), jnp.float32),
            jax.ShapeDtypeStruct((N, 8), jnp.float32),
            jax.ShapeDtypeStruct((N, 8), jnp.float32),
            jax.ShapeDtypeStruct((N, 8), jnp.float32),
            jax.ShapeDtypeStruct((N, 8), jnp.float32),
            jax.ShapeDtypeStruct((P, 8), jnp.float32),
            jax.ShapeDtypeStruct((P, 8), jnp.float32),
        ],
    )(h, W0, W1, alm0, arm0, alm1, arm1)


def _sc_body(featv, t1_hbm, mm_hbm, edges_hbm,
             z_hbm, den_hbm, ex_hbm,
             t1_sh, den_sh, acc_sh,
             srcA, srcB, srcC, dstA, dstB, dstC,
             exA, exB, exC, g1A, g1B, mb,
             semioA, semioB, semioC, semgA, semgB, semgC, semsA, semsB):
    pl.run_scoped(
        lambda rwA, rwB, rwC: _sc_inner(
            featv, t1_hbm, mm_hbm, edges_hbm, z_hbm, den_hbm, ex_hbm,
            t1_sh, den_sh, acc_sh, srcA, srcB, srcC, dstA, dstB, dstC,
            exA, exB, exC, rwA, rwB, rwC, g1A, g1B, mb,
            semioA, semioB, semioC, semgA, semgB, semgC, semsA, semsB),
        pltpu.VMEM((64, 128), jnp.float32),
        pltpu.VMEM((64, 128), jnp.float32),
        pltpu.VMEM((64, 128), jnp.float32),
    )


def _sc_inner(featv, t1_hbm, mm_hbm, edges_hbm,
              z_hbm, den_hbm, ex_hbm,
              t1_sh, den_sh, acc_sh,
              srcA, srcB, srcC, dstA, dstB, dstC,
              exA, exB, exC, rwA, rwB, rwC, g1A, g1B, mb,
              semioA, semioB, semioC, semgA, semgB, semgC, semsA, semsB):
    p = lax.axis_index("c")
    s = lax.axis_index("s")
    r0 = s * NSA
    last = s == NT - 1
    swapidx = lax.iota(jnp.int32, 16) ^ 8
    zvec = jnp.zeros((16,), jnp.float32)

    pltpu.sync_copy(t1_hbm.at[p, pl.ds(r0, NSA)], t1_sh.at[pl.ds(r0, NSA)])
    pltpu.sync_copy(mm_hbm.at[p], mb)

    @pl.when(last)
    def _tail_stage():
        pltpu.sync_copy(t1_hbm.at[p, pl.ds(N - 16, 16)],
                        t1_sh.at[pl.ds(N - 16, 16)])

    @pl.loop(0, 64)
    def _z16(b):
        exA[b, pl.ds(0, 16)] = zvec

    for c in range(9):
        pltpu.sync_copy(exA, den_sh.at[pl.ds(r0 + 64 * c, 64)])
    pltpu.sync_copy(exA.at[pl.ds(0, 48)], den_sh.at[pl.ds(r0 + 576, 48)])

    @pl.when(last)
    def _tail_den0():
        pltpu.sync_copy(exA.at[pl.ds(0, 16)], den_sh.at[pl.ds(N - 16, 16)])

    m = mb[pl.ds(0, 16)][0]
    plsc.subcore_barrier()

    eslots = ((srcA, dstA, g1A, exA, semioA, semgA, semsA),
              (srcB, dstB, g1B, exB, semioB, semgB, semsB))

    def e_io_issue(c, sl):
        sv, dv, _, _, semio, _, _ = sl
        off = 64 * c
        pltpu.async_copy(edges_hbm.at[p, 0, pl.ds(off, 64)], sv, semio)
        pltpu.async_copy(edges_hbm.at[p, 1, pl.ds(off, 64)], dv, semio)

    def e_io_wait(sl):
        sv, dv, _, _, semio, _, _ = sl
        pltpu.make_async_copy(edges_hbm.at[p, 0, pl.ds(0, 64)], sv, semio).wait()
        pltpu.make_async_copy(edges_hbm.at[p, 1, pl.ds(0, 64)], dv, semio).wait()

    def e_gather_issue(sl):
        sv, dv, g1, ex, _, semg, _ = sl
        pltpu.async_copy(t1_sh.at[sv], g1, semg)
        pltpu.async_copy(t1_sh.at[dv], ex, semg)

    def e_gather_wait(sl):
        sv, dv, g1, ex, _, semg, _ = sl
        pltpu.make_async_copy(t1_sh.at[sv], g1, semg).wait()
        pltpu.make_async_copy(t1_sh.at[dv], ex, semg).wait()

    def edge_compute(g1, g2):
        @pl.loop(0, 64)
        def _(b):
            v1 = g1[b, pl.ds(0, 16)]
            v2 = g2[b, pl.ds(0, 16)].at[swapidx].get(mode="promise_in_bounds")
            v = v1 + v2
            e = jnp.where(v > 0, v, 0.2 * v)
            g2[b, pl.ds(0, 16)] = jnp.exp(e - m)

    def e_store_issue(c, sl):
        ex, sems = sl[3], sl[6]
        pltpu.async_copy(ex, ex_hbm.at[p, pl.ds(64 * c, 64)], sems)

    def e_store_wait(sl):
        ex, sems = sl[3], sl[6]
        pltpu.make_async_copy(ex, ex_hbm.at[p, pl.ds(0, 64)], sems).wait()

    e_io_issue(s, eslots[0])
    e_io_issue(s + NT, eslots[1])
    e_io_wait(eslots[0])
    e_gather_issue(eslots[0])
    e_io_wait(eslots[1])
    e_gather_issue(eslots[1])

    @pl.loop(0, 156)
    def _edge(t):
        for x in range(2):
            sl = eslots[x]
            e_gather_wait(sl)
            edge_compute(sl[2], sl[3])
            pltpu.sync_copy(sl[3], den_sh.at[sl[1]], add=True)
            e_store_issue(s + NT * (2 * t + x), sl)

            @pl.when(t < 155)
            def _pref_io():
                e_io_issue(s + NT * (2 * (t + 1) + x), sl)

        for x in range(2):
            sl = eslots[x]
            e_store_wait(sl)

            @pl.when(t < 155)
            def _pref_g():
                e_io_wait(sl)
                e_gather_issue(sl)

    @pl.when(s < 8)
    def _edge_tail():
        c = 4992 + s
        sl = eslots[0]
        e_io_issue(c, sl)
        e_io_wait(sl)
        e_gather_issue(sl)
        e_gather_wait(sl)
        edge_compute(sl[2], sl[3])
        pltpu.sync_copy(sl[3], den_sh.at[sl[1]], add=True)
        e_store_issue(c, sl)
        e_store_wait(sl)

    plsc.subcore_barrier()

    pltpu.sync_copy(den_sh.at[pl.ds(r0, NSA)], den_hbm.at[p, pl.ds(r0, NSA)])

    @pl.when(last)
    def _tail_dend():
        pltpu.sync_copy(den_sh.at[pl.ds(N - 16, 16)],
                        den_hbm.at[p, pl.ds(N - 16, 16)])

    slots = ((srcA, dstA, exA, rwA, semioA, semgA),
             (srcB, dstB, exB, rwB, semioB, semgB),
             (srcC, dstC, exC, rwC, semioC, semgC))

    def a_io_issue(c, sl):
        sv, dv, ex, _, semio, _ = sl
        off = 64 * c
        pltpu.async_copy(edges_hbm.at[p, 0, pl.ds(off, 64)], sv, semio)
        pltpu.async_copy(edges_hbm.at[p, 1, pl.ds(off, 64)], dv, semio)
        pltpu.async_copy(ex_hbm.at[p, pl.ds(off, 64)], ex, semio)

    def a_io_wait(sl):
        sv, dv, ex, _, semio, _ = sl
        pltpu.make_async_copy(edges_hbm.at[p, 0, pl.ds(0, 64)], sv, semio).wait()
        pltpu.make_async_copy(edges_hbm.at[p, 1, pl.ds(0, 64)], dv, semio).wait()
        pltpu.make_async_copy(ex_hbm.at[p, pl.ds(0, 64)], ex, semio).wait()

    def a_gather_issue(k, sl):
        sv, _, _, rw, _, semg = sl
        for j in range(4):
            v = sv[pl.ds(16 * j, 16)]
            sv[pl.ds(16 * j, 16)] = p * (4 * N) + 4 * v + k
        pltpu.async_copy(featv.at[sv], rw, semg)

    def a_gather_wait(sl):
        sv, _, _, rw, _, semg = sl
        pltpu.make_async_copy(featv.at[sv], rw, semg).wait()

    def agg_scale(k, sl):
        _, _, ex, rw, _, _ = sl

        @pl.loop(0, 64)
        def _(b):
            exv = ex[b, pl.ds(0, 16)]
            a0 = exv[2 * k]
            a1 = exv[2 * k + 1]
            for j in range(4):
                rw[b, pl.ds(16 * j, 16)] = rw[b, pl.ds(16 * j, 16)] * a0
            for j in range(4, 8):
                rw[b, pl.ds(16 * j, 16)] = rw[b, pl.ds(16 * j, 16)] * a1

    for k in range(4):
        @pl.loop(0, 64)
        def _zr(b):
            for j in range(8):
                rwA[b, pl.ds(16 * j, 16)] = zvec

        for c in range(9):
            pltpu.sync_copy(rwA, acc_sh.at[pl.ds(r0 + 64 * c, 64)])
        pltpu.sync_copy(rwA.at[pl.ds(0, 48)], acc_sh.at[pl.ds(r0 + 576, 48)])

        @pl.when(last)
        def _tail_acc0():
            pltpu.sync_copy(rwA.at[pl.ds(0, 16)], acc_sh.at[pl.ds(N - 16, 16)])

        plsc.subcore_barrier()

        for x in range(3):
            a_io_issue(s + NT * x, slots[x])
        for x in range(3):
            a_io_wait(slots[x])
            a_gather_issue(k, slots[x])

        @pl.loop(0, 104)
        def _agg(t):
            for x in range(3):
                sl = slots[x]
                a_gather_wait(sl)
                agg_scale(k, sl)
                pltpu.sync_copy(sl[3], acc_sh.at[sl[1]], add=True)

                @pl.when(t < 103)
                def _pref_io():
                    a_io_issue(s + NT * (3 * (t + 1) + x), sl)

            for x in range(3):
                sl = slots[x]

                @pl.when(t < 103)
                def _pref_g():
                    a_io_wait(sl)
                    a_gather_issue(k, sl)

        @pl.when(s < 8)
        def _agg_tail():
            c = 4992 + s
            sl = slots[0]
            a_io_issue(c, sl)
            a_io_wait(sl)
            a_gather_issue(k, sl)
            a_gather_wait(sl)
            agg_scale(k, sl)
            pltpu.sync_copy(sl[3], acc_sh.at[sl[1]], add=True)

        plsc.subcore_barrier()
        pltpu.sync_copy(acc_sh.at[pl.ds(r0, NSA)],
                        z_hbm.at[p, pl.ds(r0, NSA), pl.ds(128 * k, 128)])

        @pl.when(last)
        def _tail_accd():
            pltpu.sync_copy(acc_sh.at[pl.ds(N - 16, 16)],
                            z_hbm.at[p, pl.ds(N - 16, 16), pl.ds(128 * k, 128)])

        plsc.subcore_barrier()

def _sc_kernel(featv, t1, mm, edges):
    mesh = plsc.VectorSubcoreMesh(core_axis_name="c", subcore_axis_name="s")
    f = pl.kernel(
        _sc_body,
        out_type=[
            jax.ShapeDtypeStruct((P, N, 512), jnp.float32),
            jax.ShapeDtypeStruct((P, N, 16), jnp.float32),
            jax.ShapeDtypeStruct((P, E, 16), jnp.float32),
        ],
        mesh=mesh,
        compiler_params=pltpu.CompilerParams(use_tc_tiling_on_sc=False),
        scratch_types=[
            pltpu.VMEM_SHARED((N, 16), jnp.float32),
            pltpu.VMEM_SHARED((N, 16), jnp.float32),
            pltpu.VMEM_SHARED((N, 128), jnp.float32),
            pltpu.VMEM((64,), jnp.int32),
            pltpu.VMEM((64,), jnp.int32),
            pltpu.VMEM((64,), jnp.int32),
            pltpu.VMEM((64,), jnp.int32),
            pltpu.VMEM((64,), jnp.int32),
            pltpu.VMEM((64,), jnp.int32),
            pltpu.VMEM((64, 16), jnp.float32),
            pltpu.VMEM((64, 16), jnp.float32),
            pltpu.VMEM((64, 16), jnp.float32),
            pltpu.VMEM((64, 16), jnp.float32),
            pltpu.VMEM((64, 16), jnp.float32),
            pltpu.VMEM((16,), jnp.float32),
            pltpu.SemaphoreType.DMA,
            pltpu.SemaphoreType.DMA,
            pltpu.SemaphoreType.DMA,
            pltpu.SemaphoreType.DMA,
            pltpu.SemaphoreType.DMA,
            pltpu.SemaphoreType.DMA,
            pltpu.SemaphoreType.DMA,
            pltpu.SemaphoreType.DMA,
        ],
    )
    return f(featv, t1, mm, edges)


def _kb_body(z_ref, den_ref, b0_ref, b1_ref, pw1_ref, pb1_ref, pw2_ref,
             out_ref):
    def path(pi, b_ref):
        zb = z_ref[pi]
        inv = 1.0 / jnp.maximum(den_ref[pi], 1e-30)
        parts = [zb[:, 64 * h:64 * (h + 1)] * inv[:, h:h + 1] for h in range(8)]
        zn = jnp.concatenate(parts, axis=1)
        x = zn + b_ref[...]
        za = jnp.where(x > 0, x, jnp.exp(jnp.minimum(x, 0.0)) - 1.0)
        t = jnp.tanh(jnp.dot(za, pw1_ref[...],
                             preferred_element_type=jnp.float32) + pb1_ref[...])
        w = jnp.sum(t * pw2_ref[...], axis=1, keepdims=True)
        return za, w

    z0, w0 = path(0, b0_ref)
    z1, w1 = path(1, b1_ref)
    wm = jnp.maximum(w0, w1)
    e0 = jnp.exp(w0 - wm)
    e1 = jnp.exp(w1 - wm)
    den = e0 + e1
    out_ref[...] = (e0 / den) * z0 + (e1 / den) * z1


def _kernel_b(z, den8, b0f, b1f, pW1, pb1, pW2t):
    BN = 1000
    grid = (N // BN,)
    full = lambda i: (0, 0)
    return pl.pallas_call(
        _kb_body,
        grid=grid,
        in_specs=[
            pl.BlockSpec((P, BN, 512), lambda i: (0, i, 0)),
            pl.BlockSpec((P, BN, 8), lambda i: (0, i, 0)),
            pl.BlockSpec((1, 512), full),
            pl.BlockSpec((1, 512), full),
            pl.BlockSpec((512, HID), full),
            pl.BlockSpec((1, HID), full),
            pl.BlockSpec((1, HID), full),
        ],
        out_specs=pl.BlockSpec((BN, 512), lambda i: (i, 0)),
        out_shape=jax.ShapeDtypeStruct((N, 512), jnp.float32),
    )(z, den8, b0f, b1f, pW1, pb1, pW2t)


def kernel(h, edge_index_0, edge_index_1, W0, al0, ar0, b0, W1, al1, ar1, b1,
           pW1, pb1, pW2):
    eye8 = jnp.eye(8, dtype=jnp.float32)
    expand = lambda a: (eye8[:, None, :] * a[:, :, None]).reshape(H * D, 8)
    alm0, arm0 = expand(al0), expand(ar0)
    alm1, arm1 = expand(al1), expand(ar1)

    f0, f1, el0, er0, el1, er1, mel, mer = _kernel_a(
        h, W0, W1, alm0, arm0, alm1, arm1)

    featv = jnp.concatenate(
        [f0.reshape(4 * N, 128), f1.reshape(4 * N, 128)], axis=0)
    el = jnp.stack([el0, el1])
    er = jnp.stack([er0, er1])
    t1 = jnp.concatenate([el, er], axis=-1)
    sm = mel[:, :1] + mer[:, :1]
    mm = jnp.broadcast_to(jnp.maximum(sm, 0.2 * sm), (P, 16))

    edges = jnp.stack([
        jnp.stack([edge_index_0[0], edge_index_0[1]]),
        jnp.stack([edge_index_1[0], edge_index_1[1]]),
    ]).astype(jnp.int32)

    z, den, _ex = _sc_kernel(featv, t1, mm, edges)

    den8 = den[:, :, :8]
    b0f = (b0.reshape(1, H * D)).astype(jnp.float32)
    b1f = (b1.reshape(1, H * D)).astype(jnp.float32)
    pb1r = pb1.reshape(1, HID)
    pW2t = pW2.reshape(1, HID)

    return _kernel_b(z, den8, b0f, b1f, pW1, pb1r, pW2t)


# speedup vs baseline: 1.3633x; 1.3633x over previous
import functools

import jax
import jax.numpy as jnp
from jax import lax
from jax.experimental import pallas as pl
from jax.experimental.pallas import tpu as pltpu
from jax.experimental.pallas import tpu_sc as plsc

N = 10000
E = 320000
IN = 128
H = 8
D = 64
HID = 128
P = 2

NT = 16
ER_ROWS = E // 128
NSA = 624


def _ka_body(h_ref, w0_ref, w1_ref, alm0_ref, arm0_ref, alm1_ref, arm1_ref,
             f0_ref, f1_ref, el0_ref, er0_ref, el1_ref, er1_ref,
             mel_ref, mer_ref):
    i = pl.program_id(0)
    hb = h_ref[...]
    f0 = jnp.dot(hb, w0_ref[...], preferred_element_type=jnp.float32)
    f1 = jnp.dot(hb, w1_ref[...], preferred_element_type=jnp.float32)
    f0_ref[...] = f0
    f1_ref[...] = f1
    el0 = jnp.dot(f0, alm0_ref[...], preferred_element_type=jnp.float32)
    er0 = jnp.dot(f0, arm0_ref[...], preferred_element_type=jnp.float32)
    el1 = jnp.dot(f1, alm1_ref[...], preferred_element_type=jnp.float32)
    er1 = jnp.dot(f1, arm1_ref[...], preferred_element_type=jnp.float32)
    el0_ref[...] = el0
    er0_ref[...] = er0
    el1_ref[...] = el1
    er1_ref[...] = er1

    @pl.when(i == 0)
    def _():
        mel_ref[...] = jnp.full((P, 8), -1e30, jnp.float32)
        mer_ref[...] = jnp.full((P, 8), -1e30, jnp.float32)

    mel_new = jnp.stack([jnp.full((8,), jnp.max(el0), jnp.float32),
                         jnp.full((8,), jnp.max(el1), jnp.float32)])
    mer_new = jnp.stack([jnp.full((8,), jnp.max(er0), jnp.float32),
                         jnp.full((8,), jnp.max(er1), jnp.float32)])
    mel_ref[...] = jnp.maximum(mel_ref[...], mel_new)
    mer_ref[...] = jnp.maximum(mer_ref[...], mer_new)


def _kernel_a(h, W0, W1, alm0, arm0, alm1, arm1):
    BN = 2000
    grid = (N // BN,)
    full = lambda i: (0, 0)
    return pl.pallas_call(
        _ka_body,
        grid=grid,
        in_specs=[
            pl.BlockSpec((BN, IN), lambda i: (i, 0)),
            pl.BlockSpec((IN, H * D), full),
            pl.BlockSpec((IN, H * D), full),
            pl.BlockSpec((H * D, 8), full),
            pl.BlockSpec((H * D, 8), full),
            pl.BlockSpec((H * D, 8), full),
            pl.BlockSpec((H * D, 8), full),
        ],
        out_specs=[
            pl.BlockSpec((BN, H * D), lambda i: (i, 0)),
            pl.BlockSpec((BN, H * D), lambda i: (i, 0)),
            pl.BlockSpec((BN, 8), lambda i: (i, 0)),
            pl.BlockSpec((BN, 8), lambda i: (i, 0)),
            pl.BlockSpec((BN, 8), lambda i: (i, 0)),
            pl.BlockSpec((BN, 8), lambda i: (i, 0)),
            pl.BlockSpec((P, 8), full),
            pl.BlockSpec((P, 8), full),
        ],
        out_shape=[
            jax.ShapeDtypeStruct((N, H * D), jnp.float32),
            jax.ShapeDtypeStruct((N, H * D), jnp.float32),
            jax.ShapeDtypeStruct((N, 8), jnp.float32),
            jax.ShapeDtypeStruct((N, 8), jnp.float32),
            jax.ShapeDtypeStruct((N, 8), jnp.float32),
            jax.ShapeDtypeStruct((N, 8), jnp.float32),
            jax.ShapeDtypeStruct((P, 8), jnp.float32),
            jax.ShapeDtypeStruct((P, 8), jnp.float32),
        ],
    )(h, W0, W1, alm0, arm0, alm1, arm1)


def _sc_body(featv, t1_hbm, mm_hbm, edges_hbm,
             z_hbm, den_hbm, ex_hbm,
             t1_sh, den_sh, acc_sh,
             srcA, srcB, srcC, dstA, dstB, dstC,
             exA, exB, exC, g1A, g1B, mb,
             semioA, semioB, semioC, semgA, semgB, semgC, semsA, semsB):
    pl.run_scoped(
        lambda rwA, rwB, rwC: _sc_inner(
            featv, t1_hbm, mm_hbm, edges_hbm, z_hbm, den_hbm, ex_hbm,
            t1_sh, den_sh, acc_sh, srcA, srcB, srcC, dstA, dstB, dstC,
            exA, exB, exC, rwA, rwB, rwC, g1A, g1B, mb,
            semioA, semioB, semioC, semgA, semgB, semgC, semsA, semsB),
        pltpu.VMEM((64, 128), jnp.float32),
        pltpu.VMEM((64, 128), jnp.float32),
        pltpu.VMEM((64, 128), jnp.float32),
    )


def _sc_inner(featv, t1_hbm, mm_hbm, edges_hbm,
              z_hbm, den_hbm, ex_hbm,
              t1_sh, den_sh, acc_sh,
              srcA, srcB, srcC, dstA, dstB, dstC,
              exA, exB, exC, rwA, rwB, rwC, g1A, g1B, mb,
              semioA, semioB, semioC, semgA, semgB, semgC, semsA, semsB):
    p = lax.axis_index("c")
    s = lax.axis_index("s")
    r0 = s * NSA
    last = s == NT - 1
    swapidx = lax.iota(jnp.int32, 16) ^ 8
    zvec = jnp.zeros((16,), jnp.float32)

    pltpu.sync_copy(t1_hbm.at[p, pl.ds(r0, NSA)], t1_sh.at[pl.ds(r0, NSA)])
    pltpu.sync_copy(mm_hbm.at[p], mb)

    @pl.when(last)
    def _tail_stage():
        pltpu.sync_copy(t1_hbm.at[p, pl.ds(N - 16, 16)],
                        t1_sh.at[pl.ds(N - 16, 16)])

    @pl.loop(0, 64)
    def _z16(b):
        exA[b, pl.ds(0, 16)] = zvec

    for c in range(9):
        pltpu.sync_copy(exA, den_sh.at[pl.ds(r0 + 64 * c, 64)])
    pltpu.sync_copy(exA.at[pl.ds(0, 48)], den_sh.at[pl.ds(r0 + 576, 48)])

    @pl.when(last)
    def _tail_den0():
        pltpu.sync_copy(exA.at[pl.ds(0, 16)], den_sh.at[pl.ds(N - 16, 16)])

    m = mb[pl.ds(0, 16)][0]
    plsc.subcore_barrier()

    eslots = ((srcA, dstA, g1A, exA, semioA, semgA, semsA),
              (srcB, dstB, g1B, exB, semioB, semgB, semsB))

    def e_io_issue(c, sl):
        sv, dv, _, _, semio, _, _ = sl
        off = 64 * c
        pltpu.async_copy(edges_hbm.at[p, 0, pl.ds(off, 64)], sv, semio)
        pltpu.async_copy(edges_hbm.at[p, 1, pl.ds(off, 64)], dv, semio)

    def e_io_wait(sl):
        sv, dv, _, _, semio, _, _ = sl
        pltpu.make_async_copy(edges_hbm.at[p, 0, pl.ds(0, 64)], sv, semio).wait()
        pltpu.make_async_copy(edges_hbm.at[p, 1, pl.ds(0, 64)], dv, semio).wait()

    def e_gather_issue(sl):
        sv, dv, g1, ex, _, semg, _ = sl
        pltpu.async_copy(t1_sh.at[sv], g1, semg)
        pltpu.async_copy(t1_sh.at[dv], ex, semg)

    def e_gather_wait(sl):
        sv, dv, g1, ex, _, semg, _ = sl
        pltpu.make_async_copy(t1_sh.at[sv], g1, semg).wait()
        pltpu.make_async_copy(t1_sh.at[dv], ex, semg).wait()

    def edge_compute(g1, g2):
        @plsc.parallel_loop(0, 64)
        def _(b):
            v1 = g1[b, pl.ds(0, 16)]
            v2 = g2[b, pl.ds(0, 16)].at[swapidx].get(mode="promise_in_bounds")
            v = v1 + v2
            e = jnp.where(v > 0, v, 0.2 * v)
            g2[b, pl.ds(0, 16)] = jnp.exp(e - m)

    def e_store_issue(c, sl):
        ex, sems = sl[3], sl[6]
        pltpu.async_copy(ex, ex_hbm.at[p, pl.ds(64 * c, 64)], sems)

    def e_store_wait(sl):
        ex, sems = sl[3], sl[6]
        pltpu.make_async_copy(ex, ex_hbm.at[p, pl.ds(0, 64)], sems).wait()

    e_io_issue(s, eslots[0])
    e_io_issue(s + NT, eslots[1])
    e_io_wait(eslots[0])
    e_gather_issue(eslots[0])
    e_io_wait(eslots[1])
    e_gather_issue(eslots[1])

    @pl.loop(0, 156)
    def _edge(t):
        for x in range(2):
            sl = eslots[x]
            e_gather_wait(sl)
            edge_compute(sl[2], sl[3])
            pltpu.sync_copy(sl[3], den_sh.at[sl[1]], add=True)
            e_store_issue(s + NT * (2 * t + x), sl)

            @pl.when(t < 155)
            def _pref_io():
                e_io_issue(s + NT * (2 * (t + 1) + x), sl)

        for x in range(2):
            sl = eslots[x]
            e_store_wait(sl)

            @pl.when(t < 155)
            def _pref_g():
                e_io_wait(sl)
                e_gather_issue(sl)

    @pl.when(s < 8)
    def _edge_tail():
        c = 4992 + s
        sl = eslots[0]
        e_io_issue(c, sl)
        e_io_wait(sl)
        e_gather_issue(sl)
        e_gather_wait(sl)
        edge_compute(sl[2], sl[3])
        pltpu.sync_copy(sl[3], den_sh.at[sl[1]], add=True)
        e_store_issue(c, sl)
        e_store_wait(sl)

    plsc.subcore_barrier()

    pltpu.sync_copy(den_sh.at[pl.ds(r0, NSA)], den_hbm.at[p, pl.ds(r0, NSA)])

    @pl.when(last)
    def _tail_dend():
        pltpu.sync_copy(den_sh.at[pl.ds(N - 16, 16)],
                        den_hbm.at[p, pl.ds(N - 16, 16)])

    slots = ((srcA, dstA, exA, rwA, semioA, semgA),
             (srcB, dstB, exB, rwB, semioB, semgB),
             (srcC, dstC, exC, rwC, semioC, semgC))

    def a_io_issue(c, sl):
        sv, dv, ex, _, semio, _ = sl
        off = 64 * c
        pltpu.async_copy(edges_hbm.at[p, 0, pl.ds(off, 64)], sv, semio)
        pltpu.async_copy(edges_hbm.at[p, 1, pl.ds(off, 64)], dv, semio)
        pltpu.async_copy(ex_hbm.at[p, pl.ds(off, 64)], ex, semio)

    def a_io_wait(sl):
        sv, dv, ex, _, semio, _ = sl
        pltpu.make_async_copy(edges_hbm.at[p, 0, pl.ds(0, 64)], sv, semio).wait()
        pltpu.make_async_copy(edges_hbm.at[p, 1, pl.ds(0, 64)], dv, semio).wait()
        pltpu.make_async_copy(ex_hbm.at[p, pl.ds(0, 64)], ex, semio).wait()

    def a_gather_issue(k, sl):
        sv, _, _, rw, _, semg = sl
        for j in range(4):
            v = sv[pl.ds(16 * j, 16)]
            sv[pl.ds(16 * j, 16)] = p * (4 * N) + 4 * v + k
        pltpu.async_copy(featv.at[sv], rw, semg)

    def a_gather_wait(sl):
        sv, _, _, rw, _, semg = sl
        pltpu.make_async_copy(featv.at[sv], rw, semg).wait()

    def agg_scale(k, sl):
        _, _, ex, rw, _, _ = sl

        @plsc.parallel_loop(0, 64)
        def _(b):
            exv = ex[b, pl.ds(0, 16)]
            a0 = exv[2 * k]
            a1 = exv[2 * k + 1]
            for j in range(4):
                rw[b, pl.ds(16 * j, 16)] = rw[b, pl.ds(16 * j, 16)] * a0
            for j in range(4, 8):
                rw[b, pl.ds(16 * j, 16)] = rw[b, pl.ds(16 * j, 16)] * a1

    for k in range(4):
        @pl.loop(0, 64)
        def _zr(b):
            for j in range(8):
                rwA[b, pl.ds(16 * j, 16)] = zvec

        for c in range(9):
            pltpu.sync_copy(rwA, acc_sh.at[pl.ds(r0 + 64 * c, 64)])
        pltpu.sync_copy(rwA.at[pl.ds(0, 48)], acc_sh.at[pl.ds(r0 + 576, 48)])

        @pl.when(last)
        def _tail_acc0():
            pltpu.sync_copy(rwA.at[pl.ds(0, 16)], acc_sh.at[pl.ds(N - 16, 16)])

        plsc.subcore_barrier()

        for x in range(3):
            a_io_issue(s + NT * x, slots[x])
        for x in range(3):
            a_io_wait(slots[x])
            a_gather_issue(k, slots[x])

        @pl.loop(0, 104)
        def _agg(t):
            for x in range(3):
                sl = slots[x]
                a_gather_wait(sl)
                agg_scale(k, sl)
                pltpu.sync_copy(sl[3], acc_sh.at[sl[1]], add=True)

                @pl.when(t < 103)
                def _pref_io():
                    a_io_issue(s + NT * (3 * (t + 1) + x), sl)

            for x in range(3):
                sl = slots[x]

                @pl.when(t < 103)
                def _pref_g():
                    a_io_wait(sl)
                    a_gather_issue(k, sl)

        @pl.when(s < 8)
        def _agg_tail():
            c = 4992 + s
            sl = slots[0]
            a_io_issue(c, sl)
            a_io_wait(sl)
            a_gather_issue(k, sl)
            a_gather_wait(sl)
            agg_scale(k, sl)
            pltpu.sync_copy(sl[3], acc_sh.at[sl[1]], add=True)

        plsc.subcore_barrier()
        pltpu.sync_copy(acc_sh.at[pl.ds(r0, NSA)],
                        z_hbm.at[p, pl.ds(r0, NSA), pl.ds(128 * k, 128)])

        @pl.when(last)
        def _tail_accd():
            pltpu.sync_copy(acc_sh.at[pl.ds(N - 16, 16)],
                            z_hbm.at[p, pl.ds(N - 16, 16), pl.ds(128 * k, 128)])

        plsc.subcore_barrier()

def _sc_kernel(featv, t1, mm, edges):
    mesh = plsc.VectorSubcoreMesh(core_axis_name="c", subcore_axis_name="s")
    f = pl.kernel(
        _sc_body,
        out_type=[
            jax.ShapeDtypeStruct((P, N, 512), jnp.float32),
            jax.ShapeDtypeStruct((P, N, 16), jnp.float32),
            jax.ShapeDtypeStruct((P, E, 16), jnp.float32),
        ],
        mesh=mesh,
        compiler_params=pltpu.CompilerParams(use_tc_tiling_on_sc=False),
        scratch_types=[
            pltpu.VMEM_SHARED((N, 16), jnp.float32),
            pltpu.VMEM_SHARED((N, 16), jnp.float32),
            pltpu.VMEM_SHARED((N, 128), jnp.float32),
            pltpu.VMEM((64,), jnp.int32),
            pltpu.VMEM((64,), jnp.int32),
            pltpu.VMEM((64,), jnp.int32),
            pltpu.VMEM((64,), jnp.int32),
            pltpu.VMEM((64,), jnp.int32),
            pltpu.VMEM((64,), jnp.int32),
            pltpu.VMEM((64, 16), jnp.float32),
            pltpu.VMEM((64, 16), jnp.float32),
            pltpu.VMEM((64, 16), jnp.float32),
            pltpu.VMEM((64, 16), jnp.float32),
            pltpu.VMEM((64, 16), jnp.float32),
            pltpu.VMEM((16,), jnp.float32),
            pltpu.SemaphoreType.DMA,
            pltpu.SemaphoreType.DMA,
            pltpu.SemaphoreType.DMA,
            pltpu.SemaphoreType.DMA,
            pltpu.SemaphoreType.DMA,
            pltpu.SemaphoreType.DMA,
            pltpu.SemaphoreType.DMA,
            pltpu.SemaphoreType.DMA,
        ],
    )
    return f(featv, t1, mm, edges)


def _kb_body(z_ref, den_ref, b0_ref, b1_ref, pw1_ref, pb1_ref, pw2_ref,
             out_ref):
    def path(pi, b_ref):
        zb = z_ref[pi]
        inv = 1.0 / jnp.maximum(den_ref[pi], 1e-30)
        parts = [zb[:, 64 * h:64 * (h + 1)] * inv[:, h:h + 1] for h in range(8)]
        zn = jnp.concatenate(parts, axis=1)
        x = zn + b_ref[...]
        za = jnp.where(x > 0, x, jnp.exp(jnp.minimum(x, 0.0)) - 1.0)
        t = jnp.tanh(jnp.dot(za, pw1_ref[...],
                             preferred_element_type=jnp.float32) + pb1_ref[...])
        w = jnp.sum(t * pw2_ref[...], axis=1, keepdims=True)
        return za, w

    z0, w0 = path(0, b0_ref)
    z1, w1 = path(1, b1_ref)
    wm = jnp.maximum(w0, w1)
    e0 = jnp.exp(w0 - wm)
    e1 = jnp.exp(w1 - wm)
    den = e0 + e1
    out_ref[...] = (e0 / den) * z0 + (e1 / den) * z1


def _kernel_b(z, den8, b0f, b1f, pW1, pb1, pW2t):
    BN = 1000
    grid = (N // BN,)
    full = lambda i: (0, 0)
    return pl.pallas_call(
        _kb_body,
        grid=grid,
        in_specs=[
            pl.BlockSpec((P, BN, 512), lambda i: (0, i, 0)),
            pl.BlockSpec((P, BN, 8), lambda i: (0, i, 0)),
            pl.BlockSpec((1, 512), full),
            pl.BlockSpec((1, 512), full),
            pl.BlockSpec((512, HID), full),
            pl.BlockSpec((1, HID), full),
            pl.BlockSpec((1, HID), full),
        ],
        out_specs=pl.BlockSpec((BN, 512), lambda i: (i, 0)),
        out_shape=jax.ShapeDtypeStruct((N, 512), jnp.float32),
    )(z, den8, b0f, b1f, pW1, pb1, pW2t)


def kernel(h, edge_index_0, edge_index_1, W0, al0, ar0, b0, W1, al1, ar1, b1,
           pW1, pb1, pW2):
    eye8 = jnp.eye(8, dtype=jnp.float32)
    expand = lambda a: (eye8[:, None, :] * a[:, :, None]).reshape(H * D, 8)
    alm0, arm0 = expand(al0), expand(ar0)
    alm1, arm1 = expand(al1), expand(ar1)

    f0, f1, el0, er0, el1, er1, mel, mer = _kernel_a(
        h, W0, W1, alm0, arm0, alm1, arm1)

    featv = jnp.concatenate(
        [f0.reshape(4 * N, 128), f1.reshape(4 * N, 128)], axis=0)
    el = jnp.stack([el0, el1])
    er = jnp.stack([er0, er1])
    t1 = jnp.concatenate([el, er], axis=-1)
    sm = mel[:, :1] + mer[:, :1]
    mm = jnp.broadcast_to(jnp.maximum(sm, 0.2 * sm), (P, 16))

    edges = jnp.stack([
        jnp.stack([edge_index_0[0], edge_index_0[1]]),
        jnp.stack([edge_index_1[0], edge_index_1[1]]),
    ]).astype(jnp.int32)

    z, den, _ex = _sc_kernel(featv, t1, mm, edges)

    den8 = den[:, :, :8]
    b0f = (b0.reshape(1, H * D)).astype(jnp.float32)
    b1f = (b1.reshape(1, H * D)).astype(jnp.float32)
    pb1r = pb1.reshape(1, HID)
    pW2t = pW2.reshape(1, HID)

    return _kernel_b(z, den8, b0f, b1f, pW1, pb1r, pW2t)


# speedup vs baseline: 1.3691x; 1.0042x over previous
import functools

import jax
import jax.numpy as jnp
from jax import lax
from jax.experimental import pallas as pl
from jax.experimental.pallas import tpu as pltpu
from jax.experimental.pallas import tpu_sc as plsc

N = 10000
E = 320000
IN = 128
H = 8
D = 64
HID = 128
P = 2

NT = 16
ER_ROWS = E // 128
NSA = 624


def _ka_body(h_ref, w0_ref, w1_ref, alm0_ref, arm0_ref, alm1_ref, arm1_ref,
             f0_ref, f1_ref, el0_ref, er0_ref, el1_ref, er1_ref,
             mel_ref, mer_ref):
    i = pl.program_id(0)
    hb = h_ref[...]
    f0 = jnp.dot(hb, w0_ref[...], preferred_element_type=jnp.float32)
    f1 = jnp.dot(hb, w1_ref[...], preferred_element_type=jnp.float32)
    f0_ref[...] = f0
    f1_ref[...] = f1
    el0 = jnp.dot(f0, alm0_ref[...], preferred_element_type=jnp.float32)
    er0 = jnp.dot(f0, arm0_ref[...], preferred_element_type=jnp.float32)
    el1 = jnp.dot(f1, alm1_ref[...], preferred_element_type=jnp.float32)
    er1 = jnp.dot(f1, arm1_ref[...], preferred_element_type=jnp.float32)
    el0_ref[...] = el0
    er0_ref[...] = er0
    el1_ref[...] = el1
    er1_ref[...] = er1

    @pl.when(i == 0)
    def _():
        mel_ref[...] = jnp.full((P, 8), -1e30, jnp.float32)
        mer_ref[...] = jnp.full((P, 8), -1e30, jnp.float32)

    mel_new = jnp.stack([jnp.full((8,), jnp.max(el0), jnp.float32),
                         jnp.full((8,), jnp.max(el1), jnp.float32)])
    mer_new = jnp.stack([jnp.full((8,), jnp.max(er0), jnp.float32),
                         jnp.full((8,), jnp.max(er1), jnp.float32)])
    mel_ref[...] = jnp.maximum(mel_ref[...], mel_new)
    mer_ref[...] = jnp.maximum(mer_ref[...], mer_new)


def _kernel_a(h, W0, W1, alm0, arm0, alm1, arm1):
    BN = 2000
    grid = (N // BN,)
    full = lambda i: (0, 0)
    return pl.pallas_call(
        _ka_body,
        grid=grid,
        in_specs=[
            pl.BlockSpec((BN, IN), lambda i: (i, 0)),
            pl.BlockSpec((IN, H * D), full),
            pl.BlockSpec((IN, H * D), full),
            pl.BlockSpec((H * D, 8), full),
            pl.BlockSpec((H * D, 8), full),
            pl.BlockSpec((H * D, 8), full),
            pl.BlockSpec((H * D, 8), full),
        ],
        out_specs=[
            pl.BlockSpec((BN, H * D), lambda i: (i, 0)),
            pl.BlockSpec((BN, H * D), lambda i: (i, 0)),
            pl.BlockSpec((BN, 8), lambda i: (i, 0)),
            pl.BlockSpec((BN, 8), lambda i: (i, 0)),
            pl.BlockSpec((BN, 8), lambda i: (i, 0)),
            pl.BlockSpec((BN, 8), lambda i: (i, 0)),
            pl.BlockSpec((P, 8), full),
            pl.BlockSpec((P, 8), full),
        ],
        out_shape=[
            jax.ShapeDtypeStruct((N, H * D), jnp.float32),
            jax.ShapeDtypeStruct((N, H * D), jnp.float32),
            jax.ShapeDtypeStruct((N, 8), jnp.float32),
            jax.ShapeDtypeStruct((N, 8), jnp.float32),
            jax.ShapeDtypeStruct((N, 8), jnp.float32),
            jax.ShapeDtypeStruct((N, 8), jnp.float32),
            jax.ShapeDtypeStruct((P, 8), jnp.float32),
            jax.ShapeDtypeStruct((P, 8), jnp.float32),
        ],
    )(h, W0, W1, alm0, arm0, alm1, arm1)


def _sc_body(featv, t1_hbm, mm_hbm, edges_hbm,
             z_hbm, den_hbm, ex_hbm,
             t1_sh, den_sh, acc_sh,
             srcA, srcB, dstA, dstB,
             exA, exB, g1A, g1B, mb,
             semioA, semioB, semgA, semgB, semsA, semsB):
    pl.run_scoped(
        lambda rwA, rwB: _sc_inner(
            featv, t1_hbm, mm_hbm, edges_hbm, z_hbm, den_hbm, ex_hbm,
            t1_sh, den_sh, acc_sh, srcA, srcB, dstA, dstB,
            exA, exB, rwA, rwB, g1A, g1B, mb,
            semioA, semioB, semgA, semgB, semsA, semsB),
        pltpu.VMEM((80, 128), jnp.float32),
        pltpu.VMEM((80, 128), jnp.float32),
    )


def _sc_inner(featv, t1_hbm, mm_hbm, edges_hbm,
              z_hbm, den_hbm, ex_hbm,
              t1_sh, den_sh, acc_sh,
              srcA, srcB, dstA, dstB,
              exA, exB, rwA, rwB, g1A, g1B, mb,
              semioA, semioB, semgA, semgB, semsA, semsB):
    p = lax.axis_index("c")
    s = lax.axis_index("s")
    r0 = s * NSA
    last = s == NT - 1
    swapidx = lax.iota(jnp.int32, 16) ^ 8
    zvec = jnp.zeros((16,), jnp.float32)

    pltpu.sync_copy(t1_hbm.at[p, pl.ds(r0, NSA)], t1_sh.at[pl.ds(r0, NSA)])
    pltpu.sync_copy(mm_hbm.at[p], mb)

    @pl.when(last)
    def _tail_stage():
        pltpu.sync_copy(t1_hbm.at[p, pl.ds(N - 16, 16)],
                        t1_sh.at[pl.ds(N - 16, 16)])

    @pl.loop(0, 80)
    def _z16(b):
        exA[b, pl.ds(0, 16)] = zvec

    for c in range(7):
        pltpu.sync_copy(exA, den_sh.at[pl.ds(r0 + 80 * c, 80)])
    pltpu.sync_copy(exA.at[pl.ds(0, 64)], den_sh.at[pl.ds(r0 + 560, 64)])

    @pl.when(last)
    def _tail_den0():
        pltpu.sync_copy(exA.at[pl.ds(0, 16)], den_sh.at[pl.ds(N - 16, 16)])

    m = mb[pl.ds(0, 16)][0]
    plsc.subcore_barrier()

    eslots = ((srcA, dstA, g1A, exA, semioA, semgA, semsA),
              (srcB, dstB, g1B, exB, semioB, semgB, semsB))

    def e_io_issue(c, sl):
        sv, dv, _, _, semio, _, _ = sl
        off = 80 * c
        pltpu.async_copy(edges_hbm.at[p, 0, pl.ds(off, 80)], sv, semio)
        pltpu.async_copy(edges_hbm.at[p, 1, pl.ds(off, 80)], dv, semio)

    def e_io_wait(sl):
        sv, dv, _, _, semio, _, _ = sl
        pltpu.make_async_copy(edges_hbm.at[p, 0, pl.ds(0, 80)], sv, semio).wait()
        pltpu.make_async_copy(edges_hbm.at[p, 1, pl.ds(0, 80)], dv, semio).wait()

    def e_gather_issue(sl):
        sv, dv, g1, ex, _, semg, _ = sl
        pltpu.async_copy(t1_sh.at[sv], g1, semg)
        pltpu.async_copy(t1_sh.at[dv], ex, semg)

    def e_gather_wait(sl):
        sv, dv, g1, ex, _, semg, _ = sl
        pltpu.make_async_copy(t1_sh.at[sv], g1, semg).wait()
        pltpu.make_async_copy(t1_sh.at[dv], ex, semg).wait()

    def edge_compute(g1, g2):
        def _body(b):
            v1 = g1[b, pl.ds(0, 16)]
            v2 = g2[b, pl.ds(0, 16)].at[swapidx].get(mode="promise_in_bounds")
            v = v1 + v2
            e = jnp.where(v > 0, v, 0.2 * v)
            g2[b, pl.ds(0, 16)] = jnp.exp(e - m)

        plsc.parallel_loop(0, 64)(_body)
        plsc.parallel_loop(64, 80)(_body)

    def e_store_issue(c, sl):
        ex, sems = sl[3], sl[6]
        pltpu.async_copy(ex, ex_hbm.at[p, pl.ds(80 * c, 80)], sems)

    def e_store_wait(sl):
        ex, sems = sl[3], sl[6]
        pltpu.make_async_copy(ex, ex_hbm.at[p, pl.ds(0, 80)], sems).wait()

    e_io_issue(s, eslots[0])
    e_io_issue(s + NT, eslots[1])
    e_io_wait(eslots[0])
    e_gather_issue(eslots[0])
    e_io_wait(eslots[1])
    e_gather_issue(eslots[1])

    @pl.loop(0, 125)
    def _edge(t):
        for x in range(2):
            sl = eslots[x]
            e_gather_wait(sl)
            edge_compute(sl[2], sl[3])
            pltpu.sync_copy(sl[3], den_sh.at[sl[1]], add=True)
            e_store_issue(s + NT * (2 * t + x), sl)

            @pl.when(t < 124)
            def _pref_io():
                e_io_issue(s + NT * (2 * (t + 1) + x), sl)

        for x in range(2):
            sl = eslots[x]
            e_store_wait(sl)

            @pl.when(t < 124)
            def _pref_g():
                e_io_wait(sl)
                e_gather_issue(sl)

    plsc.subcore_barrier()

    pltpu.sync_copy(den_sh.at[pl.ds(r0, NSA)], den_hbm.at[p, pl.ds(r0, NSA)])

    @pl.when(last)
    def _tail_dend():
        pltpu.sync_copy(den_sh.at[pl.ds(N - 16, 16)],
                        den_hbm.at[p, pl.ds(N - 16, 16)])

    slots = ((srcA, dstA, exA, rwA, semioA, semgA),
             (srcB, dstB, exB, rwB, semioB, semgB))

    def a_io_issue(c, sl):
        sv, dv, ex, _, semio, _ = sl
        off = 80 * c
        pltpu.async_copy(edges_hbm.at[p, 0, pl.ds(off, 80)], sv, semio)
        pltpu.async_copy(edges_hbm.at[p, 1, pl.ds(off, 80)], dv, semio)
        pltpu.async_copy(ex_hbm.at[p, pl.ds(off, 80)], ex, semio)

    def a_io_wait(sl):
        sv, dv, ex, _, semio, _ = sl
        pltpu.make_async_copy(edges_hbm.at[p, 0, pl.ds(0, 80)], sv, semio).wait()
        pltpu.make_async_copy(edges_hbm.at[p, 1, pl.ds(0, 80)], dv, semio).wait()
        pltpu.make_async_copy(ex_hbm.at[p, pl.ds(0, 80)], ex, semio).wait()

    def a_gather_issue(k, sl):
        sv, _, _, rw, _, semg = sl
        for j in range(5):
            v = sv[pl.ds(16 * j, 16)]
            sv[pl.ds(16 * j, 16)] = p * (4 * N) + 4 * v + k
        pltpu.async_copy(featv.at[sv], rw, semg)

    def a_gather_wait(sl):
        sv, _, _, rw, _, semg = sl
        pltpu.make_async_copy(featv.at[sv], rw, semg).wait()

    def agg_scale(k, sl):
        _, _, ex, rw, _, _ = sl

        def _body(b):
            exv = ex[b, pl.ds(0, 16)]
            a0 = exv[2 * k]
            a1 = exv[2 * k + 1]
            for j in range(4):
                rw[b, pl.ds(16 * j, 16)] = rw[b, pl.ds(16 * j, 16)] * a0
            for j in range(4, 8):
                rw[b, pl.ds(16 * j, 16)] = rw[b, pl.ds(16 * j, 16)] * a1

        plsc.parallel_loop(0, 64)(_body)
        plsc.parallel_loop(64, 80)(_body)

    for k in range(4):
        @pl.loop(0, 80)
        def _zr(b):
            for j in range(8):
                rwA[b, pl.ds(16 * j, 16)] = zvec

        for c in range(7):
            pltpu.sync_copy(rwA, acc_sh.at[pl.ds(r0 + 80 * c, 80)])
        pltpu.sync_copy(rwA.at[pl.ds(0, 64)], acc_sh.at[pl.ds(r0 + 560, 64)])

        @pl.when(last)
        def _tail_acc0():
            pltpu.sync_copy(rwA.at[pl.ds(0, 16)], acc_sh.at[pl.ds(N - 16, 16)])

        plsc.subcore_barrier()

        for x in range(2):
            a_io_issue(s + NT * x, slots[x])
        for x in range(2):
            a_io_wait(slots[x])
            a_gather_issue(k, slots[x])

        @pl.loop(0, 125)
        def _agg(t):
            for x in range(2):
                sl = slots[x]
                a_gather_wait(sl)
                agg_scale(k, sl)
                pltpu.sync_copy(sl[3], acc_sh.at[sl[1]], add=True)

                @pl.when(t < 124)
                def _pref_io():
                    a_io_issue(s + NT * (2 * (t + 1) + x), sl)

            for x in range(2):
                sl = slots[x]

                @pl.when(t < 124)
                def _pref_g():
                    a_io_wait(sl)
                    a_gather_issue(k, sl)

        plsc.subcore_barrier()
        pltpu.sync_copy(acc_sh.at[pl.ds(r0, NSA)],
                        z_hbm.at[p, pl.ds(r0, NSA), pl.ds(128 * k, 128)])

        @pl.when(last)
        def _tail_accd():
            pltpu.sync_copy(acc_sh.at[pl.ds(N - 16, 16)],
                            z_hbm.at[p, pl.ds(N - 16, 16), pl.ds(128 * k, 128)])

        plsc.subcore_barrier()

def _sc_kernel(featv, t1, mm, edges):
    mesh = plsc.VectorSubcoreMesh(core_axis_name="c", subcore_axis_name="s")
    f = pl.kernel(
        _sc_body,
        out_type=[
            jax.ShapeDtypeStruct((P, N, 512), jnp.float32),
            jax.ShapeDtypeStruct((P, N, 16), jnp.float32),
            jax.ShapeDtypeStruct((P, E, 16), jnp.float32),
        ],
        mesh=mesh,
        compiler_params=pltpu.CompilerParams(use_tc_tiling_on_sc=False),
        scratch_types=[
            pltpu.VMEM_SHARED((N, 16), jnp.float32),
            pltpu.VMEM_SHARED((N, 16), jnp.float32),
            pltpu.VMEM_SHARED((N, 128), jnp.float32),
            pltpu.VMEM((80,), jnp.int32),
            pltpu.VMEM((80,), jnp.int32),
            pltpu.VMEM((80,), jnp.int32),
            pltpu.VMEM((80,), jnp.int32),
            pltpu.VMEM((80, 16), jnp.float32),
            pltpu.VMEM((80, 16), jnp.float32),
            pltpu.VMEM((80, 16), jnp.float32),
            pltpu.VMEM((80, 16), jnp.float32),
            pltpu.VMEM((16,), jnp.float32),
            pltpu.SemaphoreType.DMA,
            pltpu.SemaphoreType.DMA,
            pltpu.SemaphoreType.DMA,
            pltpu.SemaphoreType.DMA,
            pltpu.SemaphoreType.DMA,
            pltpu.SemaphoreType.DMA,
        ],
    )
    return f(featv, t1, mm, edges)


def _kb_body(z_ref, den_ref, b0_ref, b1_ref, pw1_ref, pb1_ref, pw2_ref,
             out_ref):
    def path(pi, b_ref):
        zb = z_ref[pi]
        inv = 1.0 / jnp.maximum(den_ref[pi], 1e-30)
        parts = [zb[:, 64 * h:64 * (h + 1)] * inv[:, h:h + 1] for h in range(8)]
        zn = jnp.concatenate(parts, axis=1)
        x = zn + b_ref[...]
        za = jnp.where(x > 0, x, jnp.exp(jnp.minimum(x, 0.0)) - 1.0)
        t = jnp.tanh(jnp.dot(za, pw1_ref[...],
                             preferred_element_type=jnp.float32) + pb1_ref[...])
        w = jnp.sum(t * pw2_ref[...], axis=1, keepdims=True)
        return za, w

    z0, w0 = path(0, b0_ref)
    z1, w1 = path(1, b1_ref)
    wm = jnp.maximum(w0, w1)
    e0 = jnp.exp(w0 - wm)
    e1 = jnp.exp(w1 - wm)
    den = e0 + e1
    out_ref[...] = (e0 / den) * z0 + (e1 / den) * z1


def _kernel_b(z, den8, b0f, b1f, pW1, pb1, pW2t):
    BN = 1000
    grid = (N // BN,)
    full = lambda i: (0, 0)
    return pl.pallas_call(
        _kb_body,
        grid=grid,
        in_specs=[
            pl.BlockSpec((P, BN, 512), lambda i: (0, i, 0)),
            pl.BlockSpec((P, BN, 8), lambda i: (0, i, 0)),
            pl.BlockSpec((1, 512), full),
            pl.BlockSpec((1, 512), full),
            pl.BlockSpec((512, HID), full),
            pl.BlockSpec((1, HID), full),
            pl.BlockSpec((1, HID), full),
        ],
        out_specs=pl.BlockSpec((BN, 512), lambda i: (i, 0)),
        out_shape=jax.ShapeDtypeStruct((N, 512), jnp.float32),
    )(z, den8, b0f, b1f, pW1, pb1, pW2t)


def kernel(h, edge_index_0, edge_index_1, W0, al0, ar0, b0, W1, al1, ar1, b1,
           pW1, pb1, pW2):
    eye8 = jnp.eye(8, dtype=jnp.float32)
    expand = lambda a: (eye8[:, None, :] * a[:, :, None]).reshape(H * D, 8)
    alm0, arm0 = expand(al0), expand(ar0)
    alm1, arm1 = expand(al1), expand(ar1)

    f0, f1, el0, er0, el1, er1, mel, mer = _kernel_a(
        h, W0, W1, alm0, arm0, alm1, arm1)

    featv = jnp.concatenate(
        [f0.reshape(4 * N, 128), f1.reshape(4 * N, 128)], axis=0)
    el = jnp.stack([el0, el1])
    er = jnp.stack([er0, er1])
    t1 = jnp.concatenate([el, er], axis=-1)
    sm = mel[:, :1] + mer[:, :1]
    mm = jnp.broadcast_to(jnp.maximum(sm, 0.2 * sm), (P, 16))

    edges = jnp.stack([
        jnp.stack([edge_index_0[0], edge_index_0[1]]),
        jnp.stack([edge_index_1[0], edge_index_1[1]]),
    ]).astype(jnp.int32)

    z, den, _ex = _sc_kernel(featv, t1, mm, edges)

    den8 = den[:, :, :8]
    b0f = (b0.reshape(1, H * D)).astype(jnp.float32)
    b1f = (b1.reshape(1, H * D)).astype(jnp.float32)
    pb1r = pb1.reshape(1, HID)
    pW2t = pW2.reshape(1, HID)

    return _kernel_b(z, den8, b0f, b1f, pW1, pb1r, pW2t)
